# baseline (speedup 1.0000x reference)
import functools

import jax
import jax.numpy as jnp
from jax.experimental import pallas as pl
from jax.experimental.pallas import tpu as pltpu

_N = 4096
_R = 256


def _layer_body(z_ref, w_ref, out_ref, *, apply_exp):
    z = z_ref[...]
    if apply_exp:
        z = jnp.exp(z)

    n = z.shape[1]
    col = jax.lax.broadcasted_iota(jnp.int32, (1, n), 1)

    s_z = jnp.sum(z)
    z_max = jnp.max(z)
    j_last = jnp.max(jnp.where(z == z_max, col, -1))
    not_last = col != j_last
    big_zc = jnp.sum(jnp.where(not_last, z, 0.0))
    z_min = jnp.min(z)
    j_min_first = jnp.min(jnp.where(z == z_min, col, n))
    z_second = jnp.min(jnp.where(col == j_min_first, jnp.inf, z))

    s_w = jnp.sum(w_ref[...], axis=1, keepdims=True)
    slab_start = (j_last // 128) * 128
    slab = w_ref[:, pl.ds(slab_start, 128)]
    lane = j_last - slab_start
    col128 = jax.lax.broadcasted_iota(jnp.int32, (1, 128), 1)
    w_last = jnp.sum(jnp.where(col128 == lane, slab, 0.0),
                     axis=1, keepdims=True)
    first_cond = s_w > 1.0
    not_first = jnp.logical_not(first_cond)
    tmp = s_w * s_z / (s_w - 1.0)

    k_pos = z_min <= tmp
    k_zero = jnp.logical_not(k_pos)
    k_lt_n = jnp.logical_not(z_max <= tmp)
    k_gt_1 = z_second <= tmp

    big_wc = s_w - w_last
    use_big = first_cond | k_zero
    nonempty = (first_cond & k_pos) | (not_first & (k_zero | (k_lt_n & k_gt_1)))

    w_c = big_wc
    z_c = jnp.full_like(big_wc, big_zc)
    denom = jnp.where(nonempty, w_c - 1.0, 1.0)
    val = w_c * z_c / denom
    out_ref[...] = jnp.where(nonempty, val, jnp.inf)

    need_slow = jnp.any(not_first & k_pos)

    @pl.when(need_slow)
    def _slow():
        w = w_ref[...]
        m = z <= tmp
        a_w = jnp.sum(jnp.where(m, w, 0.0), axis=1, keepdims=True)
        a_z = jnp.sum(jnp.where(m, z, 0.0), axis=1, keepdims=True)
        z_m = jnp.max(jnp.where(m, z, -jnp.inf), axis=1, keepdims=True)
        cand = m & (z == z_m)
        j_star = jnp.max(jnp.where(cand, col, -1), axis=1, keepdims=True)
        w_star = jnp.sum(jnp.where(cand & (col == j_star), w, 0.0),
                         axis=1, keepdims=True)
        small_wc = a_w - w_star
        small_zc = a_z - z_m

        w_c2 = jnp.where(use_big, big_wc, small_wc)
        z_c2 = jnp.where(use_big, jnp.full_like(big_wc, big_zc), small_zc)
        denom2 = jnp.where(nonempty, w_c2 - 1.0, 1.0)
        val2 = w_c2 * z_c2 / denom2
        out_ref[...] = jnp.where(nonempty, val2, jnp.inf)


def _spiking_layer_pallas(z, w, apply_exp):
    n_out = w.shape[0]
    grid = (n_out // _R,)
    out = pl.pallas_call(
        functools.partial(_layer_body, apply_exp=apply_exp),
        grid=grid,
        in_specs=[
            pl.BlockSpec((1, _N), lambda i: (0, 0)),
            pl.BlockSpec((_R, _N), lambda i: (i, 0)),
        ],
        out_specs=pl.BlockSpec((_R, 1), lambda i: (i, 0)),
        out_shape=jax.ShapeDtypeStruct((n_out, 1), jnp.float32),
        compiler_params=pltpu.CompilerParams(
            dimension_semantics=("arbitrary",),
        ),
    )(z.reshape(1, _N), w)
    return out.reshape(n_out)


@jax.jit
def kernel(x, W1, W2):
    z1 = _spiking_layer_pallas(x, W1, apply_exp=True)
    z2 = _spiking_layer_pallas(z1, W2, apply_exp=False)
    return z2


# speedup vs baseline: 1.1345x; 1.1345x over previous
import functools

import jax
import jax.numpy as jnp
from jax.experimental import pallas as pl
from jax.experimental.pallas import tpu as pltpu

_N = 4096
_R = 256


def _layer_body(z_ref, w_ref, out_ref, sf_ref, si_ref, *, apply_exp):
    n = z_ref.shape[1]

    @pl.when(pl.program_id(0) == 0)
    def _z_scalars():
        z = z_ref[...]
        if apply_exp:
            z = jnp.exp(z)
        col = jax.lax.broadcasted_iota(jnp.int32, (1, n), 1)
        z_max = jnp.max(z)
        j_last = jnp.max(jnp.where(z == z_max, col, -1))
        not_last = col != j_last
        z_min = jnp.min(z)
        j_min_first = jnp.min(jnp.where(z == z_min, col, n))
        sf_ref[0] = jnp.sum(z)
        sf_ref[1] = z_max
        sf_ref[2] = jnp.sum(jnp.where(not_last, z, 0.0))
        sf_ref[3] = z_min
        sf_ref[4] = jnp.min(jnp.where(col == j_min_first, jnp.inf, z))
        si_ref[0] = j_last

    s_z = sf_ref[0]
    z_max = sf_ref[1]
    big_zc = sf_ref[2]
    z_min = sf_ref[3]
    z_second = sf_ref[4]
    j_last = si_ref[0]

    s_w = jnp.sum(w_ref[...], axis=1, keepdims=True)
    slab_start = (j_last // 128) * 128
    slab = w_ref[:, pl.ds(slab_start, 128)]
    lane = j_last - slab_start
    col128 = jax.lax.broadcasted_iota(jnp.int32, (1, 128), 1)
    w_last = jnp.sum(jnp.where(col128 == lane, slab, 0.0),
                     axis=1, keepdims=True)
    first_cond = s_w > 1.0
    not_first = jnp.logical_not(first_cond)
    tmp = s_w * s_z / (s_w - 1.0)

    k_pos = z_min <= tmp
    k_zero = jnp.logical_not(k_pos)
    k_lt_n = jnp.logical_not(z_max <= tmp)
    k_gt_1 = z_second <= tmp

    big_wc = s_w - w_last
    use_big = first_cond | k_zero
    nonempty = (first_cond & k_pos) | (not_first & (k_zero | (k_lt_n & k_gt_1)))

    w_c = big_wc
    z_c = jnp.full_like(big_wc, big_zc)
    denom = jnp.where(nonempty, w_c - 1.0, 1.0)
    val = w_c * z_c / denom
    out_ref[...] = jnp.where(nonempty, val, jnp.inf)

    need_slow = jnp.any(not_first & k_pos)

    @pl.when(need_slow)
    def _slow():
        z = z_ref[...]
        if apply_exp:
            z = jnp.exp(z)
        col = jax.lax.broadcasted_iota(jnp.int32, (1, n), 1)
        w = w_ref[...]
        m = z <= tmp
        a_w = jnp.sum(jnp.where(m, w, 0.0), axis=1, keepdims=True)
        a_z = jnp.sum(jnp.where(m, z, 0.0), axis=1, keepdims=True)
        z_m = jnp.max(jnp.where(m, z, -jnp.inf), axis=1, keepdims=True)
        cand = m & (z == z_m)
        j_star = jnp.max(jnp.where(cand, col, -1), axis=1, keepdims=True)
        w_star = jnp.sum(jnp.where(cand & (col == j_star), w, 0.0),
                         axis=1, keepdims=True)
        small_wc = a_w - w_star
        small_zc = a_z - z_m

        w_c2 = jnp.where(use_big, big_wc, small_wc)
        z_c2 = jnp.where(use_big, jnp.full_like(big_wc, big_zc), small_zc)
        denom2 = jnp.where(nonempty, w_c2 - 1.0, 1.0)
        val2 = w_c2 * z_c2 / denom2
        out_ref[...] = jnp.where(nonempty, val2, jnp.inf)


def _spiking_layer_pallas(z, w, apply_exp):
    n_out = w.shape[0]
    grid = (n_out // _R,)
    out = pl.pallas_call(
        functools.partial(_layer_body, apply_exp=apply_exp),
        grid=grid,
        in_specs=[
            pl.BlockSpec((1, _N), lambda i: (0, 0)),
            pl.BlockSpec((_R, _N), lambda i: (i, 0)),
        ],
        out_specs=pl.BlockSpec((_R, 1), lambda i: (i, 0)),
        out_shape=jax.ShapeDtypeStruct((n_out, 1), jnp.float32),
        scratch_shapes=[
            pltpu.SMEM((8,), jnp.float32),
            pltpu.SMEM((1,), jnp.int32),
        ],
        compiler_params=pltpu.CompilerParams(
            dimension_semantics=("arbitrary",),
        ),
    )(z.reshape(1, _N), w)
    return out.reshape(n_out)


@jax.jit
def kernel(x, W1, W2):
    z1 = _spiking_layer_pallas(x, W1, apply_exp=True)
    z2 = _spiking_layer_pallas(z1, W2, apply_exp=False)
    return z2


# speedup vs baseline: 1.2356x; 1.0891x over previous
import functools

import jax
import jax.numpy as jnp
from jax.experimental import pallas as pl
from jax.experimental.pallas import tpu as pltpu

_N = 4096
_R = 512


def _layer_body(z_ref, w_ref, out_ref, sf_ref, si_ref, *, apply_exp):
    n = z_ref.shape[1]

    @pl.when(pl.program_id(0) == 0)
    def _z_scalars():
        z = z_ref[...]
        if apply_exp:
            z = jnp.exp(z)
        col = jax.lax.broadcasted_iota(jnp.int32, (1, n), 1)
        z_max = jnp.max(z)
        j_last = jnp.max(jnp.where(z == z_max, col, -1))
        not_last = col != j_last
        z_min = jnp.min(z)
        j_min_first = jnp.min(jnp.where(z == z_min, col, n))
        sf_ref[0] = jnp.sum(z)
        sf_ref[1] = z_max
        sf_ref[2] = jnp.sum(jnp.where(not_last, z, 0.0))
        sf_ref[3] = z_min
        sf_ref[4] = jnp.min(jnp.where(col == j_min_first, jnp.inf, z))
        si_ref[0] = j_last

    s_z = sf_ref[0]
    z_max = sf_ref[1]
    big_zc = sf_ref[2]
    z_min = sf_ref[3]
    z_second = sf_ref[4]
    j_last = si_ref[0]

    s_w = jnp.sum(w_ref[...], axis=1, keepdims=True)
    slab_start = (j_last // 128) * 128
    slab = w_ref[:, pl.ds(slab_start, 128)]
    lane = j_last - slab_start
    col128 = jax.lax.broadcasted_iota(jnp.int32, (1, 128), 1)
    w_last = jnp.sum(jnp.where(col128 == lane, slab, 0.0),
                     axis=1, keepdims=True)
    first_cond = s_w > 1.0
    not_first = jnp.logical_not(first_cond)
    tmp = s_w * s_z / (s_w - 1.0)

    k_pos = z_min <= tmp
    k_zero = jnp.logical_not(k_pos)
    k_lt_n = jnp.logical_not(z_max <= tmp)
    k_gt_1 = z_second <= tmp

    big_wc = s_w - w_last
    use_big = first_cond | k_zero
    nonempty = (first_cond & k_pos) | (not_first & (k_zero | (k_lt_n & k_gt_1)))

    w_c = big_wc
    z_c = jnp.full_like(big_wc, big_zc)
    denom = jnp.where(nonempty, w_c - 1.0, 1.0)
    val = w_c * z_c / denom
    out_ref[...] = jnp.where(nonempty, val, jnp.inf)

    need_slow = jnp.any(not_first & k_pos)

    @pl.when(need_slow)
    def _slow():
        z = z_ref[...]
        if apply_exp:
            z = jnp.exp(z)
        col = jax.lax.broadcasted_iota(jnp.int32, (1, n), 1)
        w = w_ref[...]
        m = z <= tmp
        a_w = jnp.sum(jnp.where(m, w, 0.0), axis=1, keepdims=True)
        a_z = jnp.sum(jnp.where(m, z, 0.0), axis=1, keepdims=True)
        z_m = jnp.max(jnp.where(m, z, -jnp.inf), axis=1, keepdims=True)
        cand = m & (z == z_m)
        j_star = jnp.max(jnp.where(cand, col, -1), axis=1, keepdims=True)
        w_star = jnp.sum(jnp.where(cand & (col == j_star), w, 0.0),
                         axis=1, keepdims=True)
        small_wc = a_w - w_star
        small_zc = a_z - z_m

        w_c2 = jnp.where(use_big, big_wc, small_wc)
        z_c2 = jnp.where(use_big, jnp.full_like(big_wc, big_zc), small_zc)
        denom2 = jnp.where(nonempty, w_c2 - 1.0, 1.0)
        val2 = w_c2 * z_c2 / denom2
        out_ref[...] = jnp.where(nonempty, val2, jnp.inf)


def _spiking_layer_pallas(z, w, apply_exp):
    n_out = w.shape[0]
    grid = (n_out // _R,)
    out = pl.pallas_call(
        functools.partial(_layer_body, apply_exp=apply_exp),
        grid=grid,
        in_specs=[
            pl.BlockSpec((1, _N), lambda i: (0, 0)),
            pl.BlockSpec((_R, _N), lambda i: (i, 0)),
        ],
        out_specs=pl.BlockSpec((_R, 1), lambda i: (i, 0)),
        out_shape=jax.ShapeDtypeStruct((n_out, 1), jnp.float32),
        scratch_shapes=[
            pltpu.SMEM((8,), jnp.float32),
            pltpu.SMEM((1,), jnp.int32),
        ],
        compiler_params=pltpu.CompilerParams(
            dimension_semantics=("arbitrary",),
        ),
    )(z.reshape(1, _N), w)
    return out.reshape(n_out)


@jax.jit
def kernel(x, W1, W2):
    z1 = _spiking_layer_pallas(x, W1, apply_exp=True)
    z2 = _spiking_layer_pallas(z1, W2, apply_exp=False)
    return z2


# speedup vs baseline: 1.2691x; 1.0271x over previous
import jax
import jax.numpy as jnp
from jax.experimental import pallas as pl
from jax.experimental.pallas import tpu as pltpu

_N = 4096
_R = 512
_G = _N // _R


def _write_z_scalars(z, col, nn, sf_ref, si_ref):
    z_max = jnp.max(z)
    j_last = jnp.max(jnp.where(z == z_max, col, -1))
    z_min = jnp.min(z)
    j_min_first = jnp.min(jnp.where(z == z_min, col, nn))
    sf_ref[0] = jnp.sum(z)
    sf_ref[1] = z_max
    sf_ref[2] = jnp.sum(jnp.where(col != j_last, z, 0.0))
    sf_ref[3] = z_min
    sf_ref[4] = jnp.min(jnp.where(col == j_min_first, jnp.inf, z))
    si_ref[0] = j_last


def _fused_body(x_ref, w1_ref, w2_ref, out_ref, z1_ref, sf_ref, si_ref):
    n = _N
    t = pl.program_id(0)

    @pl.when(t == 0)
    def _scalars_l1():
        z = jnp.exp(x_ref[...])
        col = jax.lax.broadcasted_iota(jnp.int32, (1, n), 1)
        _write_z_scalars(z, col, n, sf_ref, si_ref)

    @pl.when(t == _G)
    def _scalars_l2():
        zc = z1_ref[...]
        colc = jax.lax.broadcasted_iota(jnp.int32, (n, 1), 0)
        _write_z_scalars(zc, colc, n, sf_ref, si_ref)

    s_z = sf_ref[0]
    z_max = sf_ref[1]
    big_zc = sf_ref[2]
    z_min = sf_ref[3]
    z_second = sf_ref[4]
    j_last = si_ref[0]

    def layer_step(w_ref, get_z_row, write):
        s_w = jnp.sum(w_ref[...], axis=1, keepdims=True)
        slab_start = (j_last // 128) * 128
        slab = w_ref[:, pl.ds(slab_start, 128)]
        col128 = jax.lax.broadcasted_iota(jnp.int32, (1, 128), 1)
        w_last = jnp.sum(jnp.where(col128 == j_last - slab_start, slab, 0.0),
                         axis=1, keepdims=True)
        first_cond = s_w > 1.0
        not_first = jnp.logical_not(first_cond)
        tmp = s_w * s_z / (s_w - 1.0)

        k_pos = z_min <= tmp
        k_zero = jnp.logical_not(k_pos)
        k_lt_n = jnp.logical_not(z_max <= tmp)
        k_gt_1 = z_second <= tmp

        big_wc = s_w - w_last
        use_big = first_cond | k_zero
        nonempty = (first_cond & k_pos) | (
            not_first & (k_zero | (k_lt_n & k_gt_1)))

        denom = jnp.where(nonempty, big_wc - 1.0, 1.0)
        val = big_wc * jnp.full_like(big_wc, big_zc) / denom
        write(jnp.where(nonempty, val, jnp.inf))

        need_slow = jnp.any(not_first & k_pos)

        @pl.when(need_slow)
        def _slow():
            z = get_z_row()
            col = jax.lax.broadcasted_iota(jnp.int32, (1, n), 1)
            w = w_ref[...]
            m = z <= tmp
            a_w = jnp.sum(jnp.where(m, w, 0.0), axis=1, keepdims=True)
            a_z = jnp.sum(jnp.where(m, z, 0.0), axis=1, keepdims=True)
            z_m = jnp.max(jnp.where(m, z, -jnp.inf), axis=1, keepdims=True)
            cand = m & (z == z_m)
            j_star = jnp.max(jnp.where(cand, col, -1), axis=1, keepdims=True)
            w_star = jnp.sum(jnp.where(cand & (col == j_star), w, 0.0),
                             axis=1, keepdims=True)
            small_wc = a_w - w_star
            small_zc = a_z - z_m

            w_c = jnp.where(use_big, big_wc, small_wc)
            z_c = jnp.where(use_big, jnp.full_like(big_wc, big_zc), small_zc)
            denom2 = jnp.where(nonempty, w_c - 1.0, 1.0)
            val2 = w_c * z_c / denom2
            write(jnp.where(nonempty, val2, jnp.inf))

    @pl.when(t < _G)
    def _layer1():
        def write(vals):
            z1_ref[pl.ds(t * _R, _R), :] = vals
        layer_step(w1_ref, lambda: jnp.exp(x_ref[...]), write)

    @pl.when(t >= _G)
    def _layer2():
        def write(vals):
            out_ref[...] = vals
        layer_step(w2_ref,
                   lambda: jnp.transpose(z1_ref[...], (1, 0)), write)


@jax.jit
def kernel(x, W1, W2):
    out = pl.pallas_call(
        _fused_body,
        grid=(2 * _G,),
        in_specs=[
            pl.BlockSpec((1, _N), lambda t: (0, 0)),
            pl.BlockSpec((_R, _N), lambda t: (jnp.minimum(t, _G - 1), 0)),
            pl.BlockSpec((_R, _N),
                         lambda t: (jnp.where(t < _G, 0, t - _G), 0)),
        ],
        out_specs=pl.BlockSpec((_R, 1),
                               lambda t: (jnp.where(t < _G, 0, t - _G), 0)),
        out_shape=jax.ShapeDtypeStruct((_N, 1), jnp.float32),
        scratch_shapes=[
            pltpu.VMEM((_N, 1), jnp.float32),
            pltpu.SMEM((8,), jnp.float32),
            pltpu.SMEM((1,), jnp.int32),
        ],
        compiler_params=pltpu.CompilerParams(
            dimension_semantics=("arbitrary",),
        ),
    )(x.reshape(1, _N), W1, W2)
    return out.reshape(_N)
